# initial kernel scaffold (unmeasured)
import jax
import jax.numpy as jnp
from jax import lax
from jax.experimental import pallas as pl
from jax.experimental.pallas import tpu as pltpu

N_DEV = 4
S = 2048
H = 8
DH = 128
DM = H * DH
W = 128
EXT = S + 2 * W
BQ = 256
BK = BQ + 2 * W
NBLK = S // BQ
SCALE = 0.08838834764831843
NEG = -1e9


def kernel(x, Wq, K_ext, V_ext, Wo):
    x2 = x[0].astype(jnp.bfloat16)
    k2 = K_ext[0].reshape(S, DM).astype(jnp.bfloat16)
    v2 = V_ext[0].reshape(S, DM).astype(jnp.bfloat16)
    wq = Wq.astype(jnp.bfloat16)
    wo = Wo.astype(jnp.bfloat16)

    def body(x_ref, wq_ref, k_ref, v_ref, wo_ref, out_ref,
             ext_k, ext_v, stage, q_scr, ctx_scr, mask_scr,
             send_sems, recv_sems):
        p = lax.axis_index("i")
        left = lax.rem(p + N_DEV - 1, N_DEV)
        right = lax.rem(p + 1, N_DEV)

        barrier = pltpu.get_barrier_semaphore()
        for nbr in (left, right):
            pl.semaphore_signal(barrier, inc=1, device_id=(nbr,),
                                device_id_type=pl.DeviceIdType.MESH)
        pl.semaphore_wait(barrier, 2)

        stage[0, :, :] = k_ref[0:W, :]
        stage[1, :, :] = k_ref[S - W:S, :]
        stage[2, :, :] = v_ref[0:W, :]
        stage[3, :, :] = v_ref[S - W:S, :]

        plan = [
            (0, ext_k, S + W, left),
            (1, ext_k, 0, right),
            (2, ext_v, S + W, left),
            (3, ext_v, 0, right),
        ]
        rdmas = []
        for i, (slot, dst, row0, tgt) in enumerate(plan):
            r = pltpu.make_async_remote_copy(
                src_ref=stage.at[slot],
                dst_ref=dst.at[pl.ds(row0, W), :],
                send_sem=send_sems.at[i],
                recv_sem=recv_sems.at[i],
                device_id=(tgt,),
                device_id_type=pl.DeviceIdType.MESH,
            )
            r.start()
            rdmas.append(r)

        ext_k[W:S + W, :] = k_ref[:, :]
        ext_v[W:S + W, :] = v_ref[:, :]

        q = lax.dot(x_ref[:, :], wq_ref[:, :],
                    preferred_element_type=jnp.float32)
        q_scr[:, :] = (q * SCALE).astype(jnp.bfloat16)

        ii = lax.broadcasted_iota(jnp.int32, (BQ, BK), 0)
        jj = lax.broadcasted_iota(jnp.int32, (BQ, BK), 1)
        band = (jj >= ii) & (jj - ii <= 2 * W)
        mask_scr[:, :] = jnp.where(band, 0.0, NEG)

        for r in rdmas:
            r.wait()

        is_first = (p == 0)
        is_last = (p == N_DEV - 1)
        j_last_thresh = S + W - (NBLK - 1) * BQ

        for h in range(H):
            c0 = h * DH
            for qb in range(NBLK):
                r0 = qb * BQ
                qblk = q_scr[r0:r0 + BQ, c0:c0 + DH]
                kblk = ext_k[r0:r0 + BK, c0:c0 + DH]
                vblk = ext_v[r0:r0 + BK, c0:c0 + DH]

                s = lax.dot_general(
                    qblk, kblk, (((1,), (1,)), ((), ())),
                    preferred_element_type=jnp.float32)
                s = s + mask_scr[:, :]
                if qb == 0:
                    s = s + jnp.where(is_first & (jj < W), NEG, 0.0)
                if qb == NBLK - 1:
                    s = s + jnp.where(is_last & (jj >= j_last_thresh),
                                      NEG, 0.0)

                m = jnp.max(s, axis=1, keepdims=True)
                e = jnp.exp(s - m)
                denom = jnp.sum(e, axis=1, keepdims=True)
                ctx = lax.dot_general(
                    e.astype(jnp.bfloat16), vblk,
                    (((1,), (0,)), ((), ())),
                    preferred_element_type=jnp.float32)
                ctx = ctx / denom
                ctx_scr[r0:r0 + BQ, c0:c0 + DH] = ctx.astype(jnp.bfloat16)

        out_ref[:, :] = lax.dot(ctx_scr[:, :], wo_ref[:, :],
                                preferred_element_type=jnp.float32)

    out = pl.pallas_call(
        body,
        out_shape=jax.ShapeDtypeStruct((S, DM), jnp.float32),
        in_specs=[pl.BlockSpec(memory_space=pltpu.VMEM)] * 5,
        out_specs=pl.BlockSpec(memory_space=pltpu.VMEM),
        scratch_shapes=[
            pltpu.VMEM((EXT, DM), jnp.bfloat16),
            pltpu.VMEM((EXT, DM), jnp.bfloat16),
            pltpu.VMEM((4, W, DM), jnp.bfloat16),
            pltpu.VMEM((S, DM), jnp.bfloat16),
            pltpu.VMEM((S, DM), jnp.bfloat16),
            pltpu.VMEM((BQ, BK), jnp.float32),
            pltpu.SemaphoreType.DMA((4,)),
            pltpu.SemaphoreType.DMA((4,)),
        ],
        compiler_params=pltpu.CompilerParams(collective_id=0),
    )(x2, wq, k2, v2, wo)

    return out.reshape(1, S, DM)


# baseline (device time: 85530 ns/iter reference)
import jax
import jax.numpy as jnp
from jax import lax
from jax.experimental import pallas as pl
from jax.experimental.pallas import tpu as pltpu

N_DEV = 4
S = 2048
H = 8
DH = 128
DM = H * DH
W = 128
EXT = S + 2 * W
BQ = 256
BK = BQ + 2 * W
NBLK = S // BQ
SCALE = 0.08838834764831843
NEG = -1e9


def kernel(x, Wq, K_ext, V_ext, Wo):
    x2 = x[0].astype(jnp.bfloat16)
    k2 = K_ext[0].reshape(S, DM).astype(jnp.bfloat16)
    v2 = V_ext[0].reshape(S, DM).astype(jnp.bfloat16)
    wq = Wq.astype(jnp.bfloat16)
    wo = Wo.astype(jnp.bfloat16)

    def body(x_ref, wq_ref, k_ref, v_ref, wo_ref, out_ref,
             ext_k, ext_v, stage, q_scr, ctx_scr, mask_scr,
             send_sems, recv_sems):
        p = lax.axis_index("i")
        left = lax.rem(p + N_DEV - 1, N_DEV)
        right = lax.rem(p + 1, N_DEV)

        barrier = pltpu.get_barrier_semaphore()
        for nbr in (left, right):
            pl.semaphore_signal(barrier, inc=1, device_id=(nbr,),
                                device_id_type=pl.DeviceIdType.MESH)
        pl.semaphore_wait(barrier, 2)

        stage[0, :, :] = k_ref[0:W, :]
        stage[1, :, :] = k_ref[S - W:S, :]
        stage[2, :, :] = v_ref[0:W, :]
        stage[3, :, :] = v_ref[S - W:S, :]

        plan = [
            (0, ext_k, S + W, left),
            (1, ext_k, 0, right),
            (2, ext_v, S + W, left),
            (3, ext_v, 0, right),
        ]
        rdmas = []
        for i, (slot, dst, row0, tgt) in enumerate(plan):
            r = pltpu.make_async_remote_copy(
                src_ref=stage.at[slot],
                dst_ref=dst.at[pl.ds(row0, W), :],
                send_sem=send_sems.at[i],
                recv_sem=recv_sems.at[i],
                device_id=(tgt,),
                device_id_type=pl.DeviceIdType.MESH,
            )
            r.start()
            rdmas.append(r)

        ext_k[W:S + W, :] = k_ref[:, :]
        ext_v[W:S + W, :] = v_ref[:, :]

        def q_step(rb, _):
            r0 = rb * BQ
            qv = lax.dot(x_ref[pl.ds(r0, BQ), :], wq_ref[:, :],
                         preferred_element_type=jnp.float32)
            q_scr[pl.ds(r0, BQ), :] = (qv * SCALE).astype(jnp.bfloat16)
            return 0

        lax.fori_loop(0, NBLK, q_step, 0)

        ii = lax.broadcasted_iota(jnp.int32, (BQ, BK), 0)
        jj = lax.broadcasted_iota(jnp.int32, (BQ, BK), 1)
        band = jnp.where((jj >= ii) & (jj - ii <= 2 * W), 0.0, NEG)
        is_first = (p == 0)
        is_last = (p == N_DEV - 1)
        j_last_thresh = S + W - (NBLK - 1) * BQ
        mask_scr[0, :, :] = band
        mask_scr[1, :, :] = band + jnp.where(is_first & (jj < W), NEG, 0.0)
        mask_scr[2, :, :] = band + jnp.where(
            is_last & (jj >= j_last_thresh), NEG, 0.0)

        for r in rdmas:
            r.wait()

        for h in range(H):
            c0 = h * DH

            def attn_step(qb, _, c0=c0):
                r0 = qb * BQ
                qblk = q_scr[pl.ds(r0, BQ), c0:c0 + DH]
                kblk = ext_k[pl.ds(r0, BK), c0:c0 + DH]
                vblk = ext_v[pl.ds(r0, BK), c0:c0 + DH]

                sel = jnp.where(qb == 0, 1,
                                jnp.where(qb == NBLK - 1, 2, 0))
                s = lax.dot_general(
                    qblk, kblk, (((1,), (1,)), ((), ())),
                    preferred_element_type=jnp.float32)
                s = s + mask_scr[sel, :, :]

                m = jnp.max(s, axis=1, keepdims=True)
                e = jnp.exp(s - m)
                denom = jnp.sum(e, axis=1, keepdims=True)
                ctx = lax.dot_general(
                    e.astype(jnp.bfloat16), vblk,
                    (((1,), (0,)), ((), ())),
                    preferred_element_type=jnp.float32)
                ctx = ctx / denom
                ctx_scr[pl.ds(r0, BQ), c0:c0 + DH] = ctx.astype(jnp.bfloat16)
                return 0

            lax.fori_loop(0, NBLK, attn_step, 0)

        def out_step(rb, _):
            r0 = rb * BQ
            out_ref[pl.ds(r0, BQ), :] = lax.dot(
                ctx_scr[pl.ds(r0, BQ), :], wo_ref[:, :],
                preferred_element_type=jnp.float32)
            return 0

        lax.fori_loop(0, NBLK, out_step, 0)

    out = pl.pallas_call(
        body,
        out_shape=jax.ShapeDtypeStruct((S, DM), jnp.float32),
        in_specs=[pl.BlockSpec(memory_space=pltpu.VMEM)] * 5,
        out_specs=pl.BlockSpec(memory_space=pltpu.VMEM),
        scratch_shapes=[
            pltpu.VMEM((EXT, DM), jnp.bfloat16),
            pltpu.VMEM((EXT, DM), jnp.bfloat16),
            pltpu.VMEM((4, W, DM), jnp.bfloat16),
            pltpu.VMEM((S, DM), jnp.bfloat16),
            pltpu.VMEM((S, DM), jnp.bfloat16),
            pltpu.VMEM((3, BQ, BK), jnp.float32),
            pltpu.SemaphoreType.DMA((4,)),
            pltpu.SemaphoreType.DMA((4,)),
        ],
        compiler_params=pltpu.CompilerParams(
            collective_id=0,
            vmem_limit_bytes=60 * 1024 * 1024,
        ),
    )(x2, wq, k2, v2, wo)

    return out.reshape(1, S, DM)
